# baseline (speedup 1.0000x reference)
import functools

import jax
import jax.numpy as jnp
from jax import lax
from jax.experimental import pallas as pl
from jax.experimental.pallas import tpu as pltpu
from jax.experimental.pallas import tpu_sc as plsc

NZ, NG, NP = 119, 20, 8
NB = 32
HID = 3 * NB
NOUT = 64
NCOMB = NZ * NG * NP
ROWS_PER_STEP = 3808
TAB_STEPS = NCOMB // ROWS_PER_STEP

NTOK = 16384 * 50
LANES = 128
NBLK = NTOK // LANES
NWORKERS = 32
BLK_PER_W = NBLK // NWORKERS
SB = 5
NSB = BLK_PER_W // SB


def _table_body(src_ref, gp_ref, pd_ref, w1_ref, b1_ref, w2_ref, b2_ref,
                out_ref):
    step = pl.program_id(0)
    r0 = step * ROWS_PER_STEP
    rows = r0 + lax.broadcasted_iota(jnp.int32, (ROWS_PER_STEP, 1), 0)
    z = rows // (NG * NP)
    g = (rows // NP) % NG
    p = rows % NP

    w1 = w1_ref[...]
    pz = jnp.dot(src_ref[...], w1[0:NB, :], preferred_element_type=jnp.float32)
    pg = jnp.dot(gp_ref[...], w1[NB:2 * NB, :], preferred_element_type=jnp.float32)
    pp = jnp.dot(pd_ref[...], w1[2 * NB:3 * NB, :], preferred_element_type=jnp.float32)

    ohz = (lax.broadcasted_iota(jnp.int32, (ROWS_PER_STEP, 128), 1) == z
           ).astype(jnp.float32)
    ohg = (lax.broadcasted_iota(jnp.int32, (ROWS_PER_STEP, 32), 1) == g
           ).astype(jnp.float32)
    ohp = (lax.broadcasted_iota(jnp.int32, (ROWS_PER_STEP, 8), 1) == p
           ).astype(jnp.float32)

    pre = (jnp.dot(ohz, pz, preferred_element_type=jnp.float32)
           + jnp.dot(ohg, pg, preferred_element_type=jnp.float32)
           + jnp.dot(ohp, pp, preferred_element_type=jnp.float32)
           + b1_ref[...])
    h = jnp.maximum(pre, 0.0)
    out_ref[...] = jnp.dot(h, w2_ref[...], preferred_element_type=jnp.float32) \
        + b2_ref[...]


def _build_table(src_pad, gp_pad, pd_emb, W1, b1, W2, b2):
    full = lambda s: pl.BlockSpec(s, lambda i: tuple(0 for _ in s))
    return pl.pallas_call(
        _table_body,
        grid=(TAB_STEPS,),
        in_specs=[
            full(src_pad.shape), full(gp_pad.shape), full(pd_emb.shape),
            full(W1.shape), full((1, HID)), full(W2.shape), full((1, NOUT)),
        ],
        out_specs=pl.BlockSpec((ROWS_PER_STEP, NOUT), lambda i: (i, 0)),
        out_shape=jax.ShapeDtypeStruct((NCOMB, NOUT), jnp.float32),
    )(src_pad, gp_pad, pd_emb, W1, b1.reshape(1, HID), W2,
      b2.reshape(1, NOUT))


def _ci_body(mz_ref, mg_ref, mp_ref, out_ref):
    z = jnp.clip(mz_ref[...], 0, NZ - 1)
    g = jnp.clip(mg_ref[...], 0, NG - 1)
    p = jnp.clip(mp_ref[...], 0, NP - 1)
    out_ref[...] = z * (NG * NP) + g * NP + p


def _combine_indices(mz, mg, mp):
    full = lambda: pl.BlockSpec((NBLK, LANES), lambda: (0, 0))
    return pl.pallas_call(
        _ci_body,
        in_specs=[full(), full(), full()],
        out_specs=full(),
        out_shape=jax.ShapeDtypeStruct((NBLK, LANES), jnp.int32),
    )(mz, mg, mp)


def _gather_body(ci_hbm, tab_hbm, out_hbm, idx_all, rows2,
                 sg0, sg1, ss0, ss1):
    sg = [sg0, sg1]
    ss = [ss0, ss1]
    wid = lax.axis_index("s") * 2 + lax.axis_index("c")
    base = wid * BLK_PER_W
    pltpu.sync_copy(ci_hbm.at[pl.ds(base, BLK_PER_W)], idx_all)

    def outer(t, _):
        for b in range(2):
            g = t * 2 + b
            tok0 = (base + g * SB) * LANES
            @pl.when(g >= 2)
            def _wait_prev_store():
                pltpu.make_async_copy(
                    rows2.at[b],
                    out_hbm.at[pl.ds(tok0 - 2 * SB * LANES, SB * LANES),
                               pl.ds(0, NOUT)],
                    ss[b]).wait()

            descs = [
                pltpu.async_copy(
                    tab_hbm.at[idx_all.at[g * SB + j]],
                    rows2.at[b].at[pl.ds(j * LANES, LANES)],
                    sg[b])
                for j in range(SB)
            ]
            for d in descs:
                d.wait()
            pltpu.async_copy(
                rows2.at[b],
                out_hbm.at[pl.ds(tok0, SB * LANES), pl.ds(0, NOUT)],
                ss[b])
        return ()

    lax.fori_loop(0, NSB // 2, outer, ())

    for b in range(2):
        gl = NSB - 2 + b
        tok0 = (base + gl * SB) * LANES
        pltpu.make_async_copy(
            rows2.at[b],
            out_hbm.at[pl.ds(tok0, SB * LANES), pl.ds(0, NOUT)],
            ss[b]).wait()


def _transpose_body(x_ref, out_ref):
    x = x_ref[...][:, 0:NOUT]
    out_ref[...] = jnp.transpose(x, (1, 0))[None]


def _transpose_out(x):
    return pl.pallas_call(
        _transpose_body,
        grid=(50, 16),
        in_specs=[pl.BlockSpec((8 * LANES, 128),
                               lambda l, j: (l * 16 + j, 0))],
        out_specs=pl.BlockSpec((1, NOUT, 8 * LANES), lambda l, j: (l, 0, j)),
        out_shape=jax.ShapeDtypeStruct((50, NOUT, 16384), jnp.float32),
    )(x)


def _gather(ci, table):
    mesh = plsc.VectorSubcoreMesh(core_axis_name="c", subcore_axis_name="s")
    k = functools.partial(
        pl.kernel,
        mesh=mesh,
        compiler_params=pltpu.CompilerParams(use_tc_tiling_on_sc=False),
        out_type=jax.ShapeDtypeStruct((NTOK, 128), jnp.float32),
        scratch_types=[
            pltpu.VMEM((BLK_PER_W, LANES), jnp.int32),
            pltpu.VMEM((2, SB * LANES, NOUT), jnp.float32),
            pltpu.SemaphoreType.DMA,
            pltpu.SemaphoreType.DMA,
            pltpu.SemaphoreType.DMA,
            pltpu.SemaphoreType.DMA,
        ],
    )(_gather_body)
    return k(ci, table)


def kernel(metals, mgp, mpd, src_emb, gp_emb, pd_emb, W1, b1, W2, b2):
    src_pad = jnp.zeros((128, NB), jnp.float32).at[:NZ].set(src_emb)
    gp_pad = jnp.zeros((32, NB), jnp.float32).at[:NG].set(gp_emb)

    table = _build_table(src_pad, gp_pad, pd_emb, W1, b1, W2, b2)

    mz = metals.T.reshape(NBLK, LANES).astype(jnp.int32)
    mg = mgp.T.reshape(NBLK, LANES).astype(jnp.int32)
    mp = mpd.T.reshape(NBLK, LANES).astype(jnp.int32)
    ci = _combine_indices(mz, mg, mp)

    gathered = _gather(ci, table)
    out_t = _transpose_out(gathered)
    return jnp.transpose(out_t, (2, 0, 1))


# speedup vs baseline: 1.5894x; 1.5894x over previous
import functools

import jax
import jax.numpy as jnp
from jax import lax
from jax.experimental import pallas as pl
from jax.experimental.pallas import tpu as pltpu
from jax.experimental.pallas import tpu_sc as plsc

NZ, NG, NP = 119, 20, 8
NB = 32
HID = 3 * NB
NOUT = 64
NCOMB = NZ * NG * NP
ROWS_PER_STEP = 3808
TAB_STEPS = NCOMB // ROWS_PER_STEP

NTOK = 16384 * 50
LANES = 128
NBLK = NTOK // LANES
NWORKERS = 32
BLK_PER_W = NBLK // NWORKERS
SB = 5
NSB = BLK_PER_W // SB


def _table_body(src_ref, gp_ref, pd_ref, w1_ref, b1_ref, w2_ref, b2_ref,
                out_ref):
    step = pl.program_id(0)
    r0 = step * ROWS_PER_STEP
    rows = r0 + lax.broadcasted_iota(jnp.int32, (ROWS_PER_STEP, 1), 0)
    z = rows // (NG * NP)
    g = (rows // NP) % NG
    p = rows % NP

    w1 = w1_ref[...]
    pz = jnp.dot(src_ref[...], w1[0:NB, :], preferred_element_type=jnp.float32)
    pg = jnp.dot(gp_ref[...], w1[NB:2 * NB, :], preferred_element_type=jnp.float32)
    pp = jnp.dot(pd_ref[...], w1[2 * NB:3 * NB, :], preferred_element_type=jnp.float32)

    ohz = (lax.broadcasted_iota(jnp.int32, (ROWS_PER_STEP, 128), 1) == z
           ).astype(jnp.float32)
    ohg = (lax.broadcasted_iota(jnp.int32, (ROWS_PER_STEP, 32), 1) == g
           ).astype(jnp.float32)
    ohp = (lax.broadcasted_iota(jnp.int32, (ROWS_PER_STEP, 8), 1) == p
           ).astype(jnp.float32)

    pre = (jnp.dot(ohz, pz, preferred_element_type=jnp.float32)
           + jnp.dot(ohg, pg, preferred_element_type=jnp.float32)
           + jnp.dot(ohp, pp, preferred_element_type=jnp.float32)
           + b1_ref[...])
    h = jnp.maximum(pre, 0.0)
    out_ref[...] = jnp.dot(h, w2_ref[...], preferred_element_type=jnp.float32) \
        + b2_ref[...]


def _build_table(src_pad, gp_pad, pd_emb, W1, b1, W2, b2):
    full = lambda s: pl.BlockSpec(s, lambda i: tuple(0 for _ in s))
    return pl.pallas_call(
        _table_body,
        grid=(TAB_STEPS,),
        in_specs=[
            full(src_pad.shape), full(gp_pad.shape), full(pd_emb.shape),
            full(W1.shape), full((1, HID)), full(W2.shape), full((1, NOUT)),
        ],
        out_specs=pl.BlockSpec((ROWS_PER_STEP, NOUT), lambda i: (i, 0)),
        out_shape=jax.ShapeDtypeStruct((NCOMB, NOUT), jnp.float32),
    )(src_pad, gp_pad, pd_emb, W1, b1.reshape(1, HID), W2,
      b2.reshape(1, NOUT))


def _ci_body(mz_ref, mg_ref, mp_ref, out_ref):
    z = jnp.clip(mz_ref[...], 0, NZ - 1)
    g = jnp.clip(mg_ref[...], 0, NG - 1)
    p = jnp.clip(mp_ref[...], 0, NP - 1)
    out_ref[...] = z * (NG * NP) + g * NP + p


def _combine_indices(mz, mg, mp):
    full = lambda: pl.BlockSpec((NBLK, LANES), lambda: (0, 0))
    return pl.pallas_call(
        _ci_body,
        in_specs=[full(), full(), full()],
        out_specs=full(),
        out_shape=jax.ShapeDtypeStruct((NBLK, LANES), jnp.int32),
    )(mz, mg, mp)


def _gather_body(ci_hbm, tab_hbm, out_hbm, idx_all, rows2,
                 sg0, sg1, ss0, ss1):
    sg = [sg0, sg1]
    ss = [ss0, ss1]
    wid = lax.axis_index("s") * 2 + lax.axis_index("c")
    base = wid * BLK_PER_W
    pltpu.sync_copy(ci_hbm.at[pl.ds(base, BLK_PER_W)], idx_all)

    def outer(t, _):
        for b in range(2):
            g = t * 2 + b
            tok0 = (base + g * SB) * LANES
            @pl.when(g >= 2)
            def _wait_prev_store():
                pltpu.make_async_copy(
                    rows2.at[b],
                    out_hbm.at[pl.ds(tok0 - 2 * SB * LANES, SB * LANES),
                               pl.ds(0, NOUT)],
                    ss[b]).wait()

            descs = [
                pltpu.async_copy(
                    tab_hbm.at[idx_all.at[g * SB + j]],
                    rows2.at[b].at[pl.ds(j * LANES, LANES)],
                    sg[b])
                for j in range(SB)
            ]
            for d in descs:
                d.wait()
            pltpu.async_copy(
                rows2.at[b],
                out_hbm.at[pl.ds(tok0, SB * LANES), pl.ds(0, NOUT)],
                ss[b])
        return ()

    lax.fori_loop(0, NSB // 2, outer, ())

    for b in range(2):
        gl = NSB - 2 + b
        tok0 = (base + gl * SB) * LANES
        pltpu.make_async_copy(
            rows2.at[b],
            out_hbm.at[pl.ds(tok0, SB * LANES), pl.ds(0, NOUT)],
            ss[b]).wait()


TCHUNK = 4096


def _transpose_body(x_ref, out_ref):
    out_ref[...] = jnp.transpose(x_ref[...][:, 0:NOUT], (1, 0))[None]


def _transpose_out(x):
    return pl.pallas_call(
        _transpose_body,
        grid=(50, 16384 // TCHUNK),
        in_specs=[pl.BlockSpec((TCHUNK, 128),
                               lambda l, j: (l * (16384 // TCHUNK) + j, 0))],
        out_specs=pl.BlockSpec((1, NOUT, TCHUNK), lambda l, j: (l, 0, j)),
        out_shape=jax.ShapeDtypeStruct((50, NOUT, 16384), jnp.float32),
    )(x)


def _gather(ci, table):
    mesh = plsc.VectorSubcoreMesh(core_axis_name="c", subcore_axis_name="s")
    k = functools.partial(
        pl.kernel,
        mesh=mesh,
        compiler_params=pltpu.CompilerParams(use_tc_tiling_on_sc=False),
        out_type=jax.ShapeDtypeStruct((NTOK, 128), jnp.float32),
        scratch_types=[
            pltpu.VMEM((BLK_PER_W, LANES), jnp.int32),
            pltpu.VMEM((2, SB * LANES, NOUT), jnp.float32),
            pltpu.SemaphoreType.DMA,
            pltpu.SemaphoreType.DMA,
            pltpu.SemaphoreType.DMA,
            pltpu.SemaphoreType.DMA,
        ],
    )(_gather_body)
    return k(ci, table)


def kernel(metals, mgp, mpd, src_emb, gp_emb, pd_emb, W1, b1, W2, b2):
    src_pad = jnp.zeros((128, NB), jnp.float32).at[:NZ].set(src_emb)
    gp_pad = jnp.zeros((32, NB), jnp.float32).at[:NG].set(gp_emb)

    table = _build_table(src_pad, gp_pad, pd_emb, W1, b1, W2, b2)

    mz = metals.T.reshape(NBLK, LANES).astype(jnp.int32)
    mg = mgp.T.reshape(NBLK, LANES).astype(jnp.int32)
    mp = mpd.T.reshape(NBLK, LANES).astype(jnp.int32)
    ci = _combine_indices(mz, mg, mp)

    gathered = _gather(ci, table)
    out_t = _transpose_out(gathered)
    return jnp.transpose(out_t, (2, 0, 1))
